# baseline (speedup 1.0000x reference)
import jax
import jax.numpy as jnp
from jax.experimental import pallas as pl
from jax.experimental.pallas import tpu as pltpu

_BM = 400
_CK = 2048


def _make_kernel(n, nt, nc):
    w_edge = n - (nc - 1) * _CK
    nfull = nc - 1

    def _fused_kernel(x_ref, w1_ref, w2_ref, arow_ref, achk_ref, o_ref,
                      h_ref, g_ref, pout_ref):
        u = pl.program_id(0)

        @pl.when(u == 0)
        def _prep():
            h_ref[:] = jnp.dot(x_ref[:], w1_ref[:],
                               preferred_element_type=jnp.float32)

        @pl.when(u < nt)
        def _phase1():
            ah = jnp.dot(arow_ref[:], h_ref[:],
                         preferred_element_type=jnp.float32)
            gblk = jnp.dot(jnp.maximum(ah, 0.0), w2_ref[:],
                           preferred_element_type=jnp.float32)
            g_ref[pl.ds(u * _BM, _BM), :] = gblk
            cs = (u + 1) * _BM // _CK
            pout_ref[pl.ds(u * _BM, _BM), :] = jnp.zeros(
                (_BM, gblk.shape[1]), jnp.float32)
            for c in range(nfull):
                @pl.when(c < cs)
                def _lower(c=c):
                    pout_ref[pl.ds(u * _BM, _BM), :] += jnp.dot(
                        arow_ref[:, c * _CK:(c + 1) * _CK],
                        g_ref[c * _CK:(c + 1) * _CK, :],
                        preferred_element_type=jnp.float32)

        @pl.when(u >= nt)
        def _phase2():
            v = u - nt
            t = v // nc
            c = v % nc
            cs = (t + 1) * _BM // _CK

            @pl.when(c == 0)
            def _init():
                o_ref[:] = pout_ref[pl.ds(t * _BM, _BM), :]

            @pl.when((c >= cs) & (c < nc - 1))
            def _upper():
                o_ref[:] += jnp.dot(achk_ref[:],
                                    g_ref[pl.ds(c * _CK, _CK), :],
                                    preferred_element_type=jnp.float32)

            @pl.when(c == nc - 1)
            def _edge():
                o_ref[:] += jnp.dot(achk_ref[:, :w_edge],
                                    g_ref[pl.ds((nc - 1) * _CK, w_edge), :],
                                    preferred_element_type=jnp.float32)

    return _fused_kernel


def kernel(x, adj_low, adj_high, W1, W2):
    n, nfeat = x.shape
    nhid = W1.shape[1]
    ncls = W2.shape[1]
    nt = n // _BM
    nc = -(-n // _CK)

    def _arow_index(u):
        return (jnp.minimum(u, nt - 1), 0)

    def _achk_index(u):
        v = jnp.maximum(u - nt, 0)
        t = v // nc
        c = v % nc
        cs = (t + 1) * _BM // _CK
        return (t, jnp.maximum(c, cs))

    def _out_index(u):
        return (jnp.maximum(u - nt, 0) // nc, 0)

    out = pl.pallas_call(
        _make_kernel(n, nt, nc),
        grid=(nt + nt * nc,),
        in_specs=[
            pl.BlockSpec((n, nfeat), lambda u: (0, 0)),
            pl.BlockSpec((nfeat, nhid), lambda u: (0, 0)),
            pl.BlockSpec((nhid, ncls), lambda u: (0, 0)),
            pl.BlockSpec((_BM, n), _arow_index),
            pl.BlockSpec((_BM, _CK), _achk_index),
        ],
        out_specs=pl.BlockSpec((_BM, ncls), _out_index),
        out_shape=jax.ShapeDtypeStruct((n, ncls), jnp.float32),
        scratch_shapes=[
            pltpu.VMEM((n, nhid), jnp.float32),
            pltpu.VMEM((n, ncls), jnp.float32),
            pltpu.VMEM((n, ncls), jnp.float32),
        ],
        compiler_params=pltpu.CompilerParams(
            dimension_semantics=("arbitrary",)),
    )(x, W1, W2, adj_low, adj_low)
    return out


# speedup vs baseline: 1.0209x; 1.0209x over previous
import jax
import jax.numpy as jnp
from jax.experimental import pallas as pl
from jax.experimental.pallas import tpu as pltpu

_BM = 400
_CK = 4096


def _make_kernel(n, nt, nc):
    w_edge = n - (nc - 1) * _CK
    nfull = nc - 1

    def _fused_kernel(x_ref, w1_ref, w2_ref, arow_ref, achk_ref, o_ref,
                      h_ref, g_ref, pout_ref):
        u = pl.program_id(0)

        @pl.when(u == 0)
        def _prep():
            h_ref[:] = jnp.dot(x_ref[:], w1_ref[:],
                               preferred_element_type=jnp.float32)

        @pl.when(u < nt)
        def _phase1():
            ah = jnp.dot(arow_ref[:], h_ref[:],
                         preferred_element_type=jnp.float32)
            gblk = jnp.dot(jnp.maximum(ah, 0.0), w2_ref[:],
                           preferred_element_type=jnp.float32)
            g_ref[pl.ds(u * _BM, _BM), :] = gblk
            cs = (u + 1) * _BM // _CK
            pout_ref[pl.ds(u * _BM, _BM), :] = jnp.zeros(
                (_BM, gblk.shape[1]), jnp.float32)
            for c in range(nfull):
                @pl.when(c < cs)
                def _lower(c=c):
                    pout_ref[pl.ds(u * _BM, _BM), :] += jnp.dot(
                        arow_ref[:, c * _CK:(c + 1) * _CK],
                        g_ref[c * _CK:(c + 1) * _CK, :],
                        preferred_element_type=jnp.float32)

        @pl.when(u >= nt)
        def _phase2():
            v = u - nt
            t = v // nc
            c = v % nc
            cs = (t + 1) * _BM // _CK

            @pl.when(c == 0)
            def _init():
                o_ref[:] = pout_ref[pl.ds(t * _BM, _BM), :]

            @pl.when((c >= cs) & (c < nc - 1))
            def _upper():
                o_ref[:] += jnp.dot(achk_ref[:],
                                    g_ref[pl.ds(c * _CK, _CK), :],
                                    preferred_element_type=jnp.float32)

            @pl.when(c == nc - 1)
            def _edge():
                o_ref[:] += jnp.dot(achk_ref[:, :w_edge],
                                    g_ref[pl.ds((nc - 1) * _CK, w_edge), :],
                                    preferred_element_type=jnp.float32)

    return _fused_kernel


def kernel(x, adj_low, adj_high, W1, W2):
    n, nfeat = x.shape
    nhid = W1.shape[1]
    ncls = W2.shape[1]
    nt = n // _BM
    nc = -(-n // _CK)

    def _arow_index(u):
        return (jnp.minimum(u, nt - 1), 0)

    def _achk_index(u):
        v = jnp.maximum(u - nt, 0)
        t = v // nc
        c = v % nc
        cs = (t + 1) * _BM // _CK
        return (t, jnp.maximum(c, cs))

    def _out_index(u):
        return (jnp.maximum(u - nt, 0) // nc, 0)

    out = pl.pallas_call(
        _make_kernel(n, nt, nc),
        grid=(nt + nt * nc,),
        in_specs=[
            pl.BlockSpec((n, nfeat), lambda u: (0, 0)),
            pl.BlockSpec((nfeat, nhid), lambda u: (0, 0)),
            pl.BlockSpec((nhid, ncls), lambda u: (0, 0)),
            pl.BlockSpec((_BM, n), _arow_index),
            pl.BlockSpec((_BM, _CK), _achk_index),
        ],
        out_specs=pl.BlockSpec((_BM, ncls), _out_index),
        out_shape=jax.ShapeDtypeStruct((n, ncls), jnp.float32),
        scratch_shapes=[
            pltpu.VMEM((n, nhid), jnp.float32),
            pltpu.VMEM((n, ncls), jnp.float32),
            pltpu.VMEM((n, ncls), jnp.float32),
        ],
        compiler_params=pltpu.CompilerParams(
            dimension_semantics=("arbitrary",),
            vmem_limit_bytes=100 * 1024 * 1024),
    )(x, W1, W2, adj_low, adj_low)
    return out


# speedup vs baseline: 1.1597x; 1.1359x over previous
import jax
import jax.numpy as jnp
from jax.experimental import pallas as pl
from jax.experimental.pallas import tpu as pltpu

_BM = 200
_CK = 2048
_BS = 1000


def _build_schedule(n, nt, nc):
    def rows_cov(c):
        return ((c + 1) * _CK // _BM) * _BM if c < nc - 1 else n

    def ready(c):
        if c < nc - 1:
            return -(-((c + 1) * _CK) // _BM) - 1
        return nt - 1

    sched = {}
    nf = 0
    for c in range(nc):
        assert rows_cov(c) % _BS == 0
        for s in range(rows_cov(c) // _BS):
            slot = max(ready(c), nf)
            nf = slot + 1
            sched[slot] = (s, c)
    glen = max(nt, nf)
    tail0 = glen - n // _BS
    assert sched[tail0][1] == nc - 1 and sched[tail0][0] == 0
    sact = [0] * glen
    srow = [0] * glen
    schk = [0] * glen
    nxt = (0, 0)
    for u in reversed(range(glen)):
        if u in sched:
            nxt = sched[u]
            sact[u] = 1
        srow[u], schk[u] = nxt
    return glen, tail0, sact, srow, schk


def _make_kernel(n, nt, nc, tail0):
    w_edge = n - (nc - 1) * _CK
    nfull = nc - 1

    def _fused_kernel(sact_ref, srow_ref, schk_ref,
                      x_ref, w1_ref, w2_ref, arow_ref, astrip_ref,
                      o_ref, h_ref, g_ref, oacc_ref):
        u = pl.program_id(0)

        @pl.when(u == 0)
        def _prep():
            h_ref[:] = jnp.dot(x_ref[:], w1_ref[:],
                               preferred_element_type=jnp.float32)

        @pl.when(u < nt)
        def _row():
            ah = jnp.dot(arow_ref[:], h_ref[:],
                         preferred_element_type=jnp.float32)
            gblk = jnp.dot(jnp.maximum(ah, 0.0), w2_ref[:],
                           preferred_element_type=jnp.float32)
            g_ref[pl.ds(u * _BM, _BM), :] = gblk
            cs = (u + 1) * _BM // _CK
            oacc_ref[pl.ds(u * _BM, _BM), :] = jnp.zeros(
                (_BM, gblk.shape[1]), jnp.float32)
            for c in range(nfull):
                @pl.when(c < cs)
                def _lower(c=c):
                    oacc_ref[pl.ds(u * _BM, _BM), :] += jnp.dot(
                        arow_ref[:, c * _CK:(c + 1) * _CK],
                        g_ref[c * _CK:(c + 1) * _CK, :],
                        preferred_element_type=jnp.float32)

        @pl.when(sact_ref[u] == 1)
        def _strip():
            r = srow_ref[u]
            c = schk_ref[u]

            @pl.when(c < nc - 1)
            def _full():
                oacc_ref[pl.ds(r * _BS, _BS), :] += jnp.dot(
                    astrip_ref[:],
                    g_ref[pl.ds(c * _CK, _CK), :],
                    preferred_element_type=jnp.float32)

            @pl.when(c == nc - 1)
            def _edge():
                oacc_ref[pl.ds(r * _BS, _BS), :] += jnp.dot(
                    astrip_ref[:, :w_edge],
                    g_ref[pl.ds((nc - 1) * _CK, w_edge), :],
                    preferred_element_type=jnp.float32)

        @pl.when(u >= tail0)
        def _flush():
            o_ref[:] = oacc_ref[pl.ds((u - tail0) * _BS, _BS), :]

    return _fused_kernel


def kernel(x, adj_low, adj_high, W1, W2):
    n, nfeat = x.shape
    nhid = W1.shape[1]
    ncls = W2.shape[1]
    nt = n // _BM
    nc = -(-n // _CK)

    glen, tail0, sact, srow, schk = _build_schedule(n, nt, nc)

    def _arow_index(u, sact_r, srow_r, schk_r):
        return (jnp.minimum(u, nt - 1), 0)

    def _astrip_index(u, sact_r, srow_r, schk_r):
        return (srow_r[u], schk_r[u])

    def _out_index(u, sact_r, srow_r, schk_r):
        return (jnp.maximum(u - tail0, 0), 0)

    def _const_index(u, sact_r, srow_r, schk_r):
        return (0, 0)

    grid_spec = pltpu.PrefetchScalarGridSpec(
        num_scalar_prefetch=3,
        grid=(glen,),
        in_specs=[
            pl.BlockSpec((n, nfeat), _const_index),
            pl.BlockSpec((nfeat, nhid), _const_index),
            pl.BlockSpec((nhid, ncls), _const_index),
            pl.BlockSpec((_BM, n), _arow_index),
            pl.BlockSpec((_BS, _CK), _astrip_index),
        ],
        out_specs=pl.BlockSpec((_BS, ncls), _out_index),
        scratch_shapes=[
            pltpu.VMEM((n, nhid), jnp.float32),
            pltpu.VMEM((n, ncls), jnp.float32),
            pltpu.VMEM((n, ncls), jnp.float32),
        ],
    )

    out = pl.pallas_call(
        _make_kernel(n, nt, nc, tail0),
        grid_spec=grid_spec,
        out_shape=jax.ShapeDtypeStruct((n, ncls), jnp.float32),
        compiler_params=pltpu.CompilerParams(
            dimension_semantics=("arbitrary",),
            vmem_limit_bytes=100 * 1024 * 1024),
    )(jnp.asarray(sact, jnp.int32), jnp.asarray(srow, jnp.int32),
      jnp.asarray(schk, jnp.int32), x, W1, W2, adj_low, adj_low)
    return out
